# baseline (speedup 1.0000x reference)
import functools

import jax
import jax.numpy as jnp
from jax import lax
from jax.experimental import pallas as pl
from jax.experimental.pallas import tpu as pltpu
from jax.experimental.pallas import tpu_sc as plsc

H = 128
N_VAR = 10000
N_PAD = 10112
CHUNK = 64
SEG = 16
CHUNK2 = 128
ZROWS = 64
ZTAIL = 56
NC = 2
NS = 16
NW = NC * NS
RPT = N_PAD // NS
NZB = RPT // ZROWS


def _sc_row_sums(x_con, src3, dst3, tpc):
    mesh = plsc.VectorSubcoreMesh(core_axis_name="c", subcore_axis_name="s")

    @functools.partial(
        pl.kernel,
        mesh=mesh,
        out_type=jax.ShapeDtypeStruct((NC, N_PAD, H), jnp.float32),
        scratch_types=[
            pltpu.VMEM_SHARED((N_PAD, H), jnp.float32),
            pltpu.VMEM((1, CHUNK), jnp.int32),
            pltpu.VMEM((1, CHUNK), jnp.int32),
            pltpu.VMEM((tpc, CHUNK), jnp.int32),
            pltpu.VMEM((CHUNK, H), jnp.float32),
            pltpu.VMEM((CHUNK, H), jnp.float32),
            pltpu.SemaphoreType.DMA,
            pltpu.SemaphoreType.DMA,
            pltpu.SemaphoreType.DMA,
            pltpu.SemaphoreType.DMA,
        ],
    )
    def k(x_hbm, src_hbm, dst_hbm, s_out, acc_sh, src_va, src_vb, dst_all,
          rows_v, rows_w, sem, sem2, sem3, sem4):
        cid = lax.axis_index("c")
        sid = lax.axis_index("s")
        wid = sid * NC + cid
        base_r = sid * RPT

        zeros16 = jnp.zeros((16,), jnp.float32)

        def fill_zero(r, carry):
            for cc in range(H // 16):
                rows_v[r, pl.ds(cc * 16, 16)] = zeros16
            return carry
        lax.fori_loop(0, CHUNK, fill_zero, 0)

        for i in range(NZB):
            pltpu.sync_copy(rows_v, acc_sh.at[pl.ds(base_r + i * ZROWS, ZROWS)])
        pltpu.sync_copy(rows_v.at[pl.ds(0, ZTAIL)],
                        acc_sh.at[pl.ds(base_r + NZB * ZROWS, ZTAIL)])

        pltpu.sync_copy(dst_hbm.at[wid], dst_all)

        plsc.subcore_barrier()

        def body(j, carry):
            pltpu.sync_copy(src_hbm.at[wid, 2 * j], src_va)
            pltpu.sync_copy(src_hbm.at[wid, 2 * j + 1], src_vb)
            ga = pltpu.async_copy(x_hbm.at[src_va.at[0]], rows_v, sem)
            ga.wait()
            sa = pltpu.async_copy(rows_v, acc_sh.at[dst_all.at[2 * j]],
                                  sem3, add=True)
            gb = pltpu.async_copy(x_hbm.at[src_vb.at[0]], rows_w, sem2)
            gb.wait()
            sa.wait()
            sb = pltpu.async_copy(rows_w, acc_sh.at[dst_all.at[2 * j + 1]],
                                  sem4, add=True)
            sb.wait()
            return carry
        lax.fori_loop(0, tpc // 2, body, 0)
        if tpc % 2:
            pltpu.sync_copy(src_hbm.at[wid, tpc - 1], src_va)
            pltpu.async_copy(x_hbm.at[src_va.at[0]], rows_v, sem).wait()
            pltpu.sync_copy(rows_v, acc_sh.at[dst_all.at[tpc - 1]], add=True)

        plsc.subcore_barrier()

        for i in range(NZB):
            r = base_r + i * ZROWS
            pltpu.sync_copy(acc_sh.at[pl.ds(r, ZROWS)], rows_v)
            pltpu.sync_copy(rows_v, s_out.at[cid, pl.ds(r, ZROWS)])
        r = base_r + NZB * ZROWS
        pltpu.sync_copy(acc_sh.at[pl.ds(r, ZTAIL)], rows_v.at[pl.ds(0, ZTAIL)])
        pltpu.sync_copy(rows_v.at[pl.ds(0, ZTAIL)], s_out.at[cid, pl.ds(r, ZTAIL)])

    return k(x_con, src3, dst3)


def _sc_counts(dst3, tpc):
    mesh = plsc.VectorSubcoreMesh(core_axis_name="c", subcore_axis_name="s")

    @functools.partial(
        pl.kernel,
        mesh=mesh,
        out_type=jax.ShapeDtypeStruct((NC, N_PAD, H), jnp.float32),
        scratch_types=[
            pltpu.VMEM_SHARED((N_PAD, H), jnp.float32),
            pltpu.VMEM((tpc, CHUNK2), jnp.int32),
            pltpu.VMEM((CHUNK2, H), jnp.float32),
        ],
    )
    def k(dst_hbm, c_out, cnt_sh, dst_all, ones_v):
        cid = lax.axis_index("c")
        sid = lax.axis_index("s")
        wid = sid * NC + cid
        base_r = sid * RPT

        zeros16 = jnp.zeros((16,), jnp.float32)
        ones16 = jnp.ones((16,), jnp.float32)

        def fill_zero(r, carry):
            for cc in range(H // 16):
                ones_v[r, pl.ds(cc * 16, 16)] = zeros16
            return carry
        lax.fori_loop(0, CHUNK2, fill_zero, 0)

        for i in range(NZB):
            pltpu.sync_copy(ones_v.at[pl.ds(0, ZROWS)],
                            cnt_sh.at[pl.ds(base_r + i * ZROWS, ZROWS)])
        pltpu.sync_copy(ones_v.at[pl.ds(0, ZTAIL)],
                        cnt_sh.at[pl.ds(base_r + NZB * ZROWS, ZTAIL)])

        def fill_one(r, carry):
            for cc in range(H // 16):
                ones_v[r, pl.ds(cc * 16, 16)] = ones16
            return carry
        lax.fori_loop(0, CHUNK2, fill_one, 0)

        pltpu.sync_copy(dst_hbm.at[wid], dst_all)

        plsc.subcore_barrier()

        def body(j, carry):
            pltpu.sync_copy(ones_v, cnt_sh.at[dst_all.at[j]], add=True)
            return carry
        lax.fori_loop(0, tpc, body, 0)

        plsc.subcore_barrier()

        for i in range(NZB):
            r = base_r + i * ZROWS
            pltpu.sync_copy(cnt_sh.at[pl.ds(r, ZROWS)], ones_v.at[pl.ds(0, ZROWS)])
            pltpu.sync_copy(ones_v.at[pl.ds(0, ZROWS)], c_out.at[cid, pl.ds(r, ZROWS)])
        r = base_r + NZB * ZROWS
        pltpu.sync_copy(cnt_sh.at[pl.ds(r, ZTAIL)], ones_v.at[pl.ds(0, ZTAIL)])
        pltpu.sync_copy(ones_v.at[pl.ds(0, ZTAIL)], c_out.at[cid, pl.ds(r, ZTAIL)])

    return k(dst3)


def _tc_finish(sa, sb, ca, cb, W, b, gamma, beta):
    R = 1000

    def body(sa_ref, sb_ref, ca_ref, cb_ref, w_ref, b_ref, g_ref, be_ref, o_ref):
        s = sa_ref[...] + sb_ref[...]
        cnt = ca_ref[...] + cb_ref[...]
        y = lax.dot_general(s, w_ref[...], (((1,), (1,)), ((), ())),
                            preferred_element_type=jnp.float32)
        agg = (y + cnt * b_ref[...]) / (cnt + 1e-6)
        h = jnp.maximum(agg, 0.0)
        mean = jnp.mean(h, axis=1, keepdims=True)
        cen = h - mean
        var = jnp.mean(cen * cen, axis=1, keepdims=True)
        o_ref[...] = cen * lax.rsqrt(var + 1e-5) * g_ref[...] + be_ref[...]

    return pl.pallas_call(
        body,
        grid=(N_VAR // R,),
        in_specs=[
            pl.BlockSpec((R, H), lambda i: (i, 0)),
            pl.BlockSpec((R, H), lambda i: (i, 0)),
            pl.BlockSpec((R, H), lambda i: (i, 0)),
            pl.BlockSpec((R, H), lambda i: (i, 0)),
            pl.BlockSpec((H, H), lambda i: (0, 0)),
            pl.BlockSpec((1, H), lambda i: (0, 0)),
            pl.BlockSpec((1, H), lambda i: (0, 0)),
            pl.BlockSpec((1, H), lambda i: (0, 0)),
        ],
        out_specs=pl.BlockSpec((R, H), lambda i: (i, 0)),
        out_shape=jax.ShapeDtypeStruct((N_VAR, H), jnp.float32),
    )(sa, sb, ca, cb, W, b.reshape(1, H), gamma.reshape(1, H), beta.reshape(1, H))


def kernel(x_con, edge_index, num_var, W, b, gamma, beta):
    src = edge_index[0].astype(jnp.int32)
    dst = jnp.minimum(edge_index[1], num_var - 1).astype(jnp.int32)

    e = src.shape[0]
    e_pad = -e % (CHUNK * NW)
    src1 = jnp.concatenate([src, jnp.zeros((e_pad,), jnp.int32)]) if e_pad else src
    dst1 = jnp.concatenate([dst, jnp.full((e_pad,), N_PAD - 1, jnp.int32)]) if e_pad else dst
    tpc = (e + e_pad) // (CHUNK * NW)
    src3 = src1.reshape(NW, tpc, 1, CHUNK)
    dst3 = dst1.reshape(NW, tpc, CHUNK)

    e_pad2 = -e % (CHUNK2 * NW)
    dst2 = jnp.concatenate([dst, jnp.full((e_pad2,), N_PAD - 1, jnp.int32)]) if e_pad2 else dst
    tpc2 = (e + e_pad2) // (CHUNK2 * NW)
    dst3b = dst2.reshape(NW, tpc2, CHUNK2)

    s_part = _sc_row_sums(x_con, src3, dst3, tpc)
    c_part = _sc_counts(dst3b, tpc2)
    return _tc_finish(s_part[0, :N_VAR], s_part[1, :N_VAR],
                      c_part[0, :N_VAR], c_part[1, :N_VAR],
                      W, b, gamma, beta)


# speedup vs baseline: 1.0967x; 1.0967x over previous
import functools

import jax
import jax.numpy as jnp
from jax import lax
from jax.experimental import pallas as pl
from jax.experimental.pallas import tpu as pltpu
from jax.experimental.pallas import tpu_sc as plsc

H = 128
N_VAR = 10000
N_PAD = 10112
CHUNK = 64
SEG = 16
CHUNK2 = 128
ZROWS = 64
ZTAIL = 56
NC = 2
NS = 16
NW = NC * NS
RPT = N_PAD // NS
NZB = RPT // ZROWS


def _sc_row_sums(x_con, src3, dst3, tpc):
    mesh = plsc.VectorSubcoreMesh(core_axis_name="c", subcore_axis_name="s")

    @functools.partial(
        pl.kernel,
        mesh=mesh,
        out_type=jax.ShapeDtypeStruct((NC, N_PAD, H), jnp.float32),
        scratch_types=[
            pltpu.VMEM_SHARED((N_PAD, H), jnp.float32),
            pltpu.VMEM((tpc, CHUNK), jnp.int32),
            pltpu.VMEM((tpc, CHUNK), jnp.int32),
            pltpu.VMEM((CHUNK, H), jnp.float32),
            pltpu.SemaphoreType.DMA,
        ],
    )
    def k(x_hbm, src_hbm, dst_hbm, s_out, acc_sh, src_all, dst_all,
          rows_v, sem):
        cid = lax.axis_index("c")
        sid = lax.axis_index("s")
        wid = sid * NC + cid
        base_r = sid * RPT

        zeros16 = jnp.zeros((16,), jnp.float32)

        def fill_zero(r, carry):
            for cc in range(H // 16):
                rows_v[r, pl.ds(cc * 16, 16)] = zeros16
            return carry
        lax.fori_loop(0, CHUNK, fill_zero, 0)

        for i in range(NZB):
            pltpu.sync_copy(rows_v, acc_sh.at[pl.ds(base_r + i * ZROWS, ZROWS)])
        pltpu.sync_copy(rows_v.at[pl.ds(0, ZTAIL)],
                        acc_sh.at[pl.ds(base_r + NZB * ZROWS, ZTAIL)])

        pltpu.sync_copy(src_hbm.at[wid], src_all)
        pltpu.sync_copy(dst_hbm.at[wid], dst_all)

        plsc.subcore_barrier()

        def body(j, carry):
            pltpu.async_copy(x_hbm.at[src_all.at[j]], rows_v, sem).wait()
            pltpu.sync_copy(rows_v, acc_sh.at[dst_all.at[j]], add=True)
            return carry
        lax.fori_loop(0, tpc, body, 0)

        plsc.subcore_barrier()

        for i in range(NZB):
            r = base_r + i * ZROWS
            pltpu.sync_copy(acc_sh.at[pl.ds(r, ZROWS)], rows_v)
            pltpu.sync_copy(rows_v, s_out.at[cid, pl.ds(r, ZROWS)])
        r = base_r + NZB * ZROWS
        pltpu.sync_copy(acc_sh.at[pl.ds(r, ZTAIL)], rows_v.at[pl.ds(0, ZTAIL)])
        pltpu.sync_copy(rows_v.at[pl.ds(0, ZTAIL)], s_out.at[cid, pl.ds(r, ZTAIL)])

    return k(x_con, src3, dst3)


def _sc_counts(dst3, tpc):
    mesh = plsc.VectorSubcoreMesh(core_axis_name="c", subcore_axis_name="s")

    @functools.partial(
        pl.kernel,
        mesh=mesh,
        out_type=jax.ShapeDtypeStruct((NC, N_PAD, H), jnp.float32),
        scratch_types=[
            pltpu.VMEM_SHARED((N_PAD, H), jnp.float32),
            pltpu.VMEM((tpc, CHUNK2), jnp.int32),
            pltpu.VMEM((CHUNK2, H), jnp.float32),
        ],
    )
    def k(dst_hbm, c_out, cnt_sh, dst_all, ones_v):
        cid = lax.axis_index("c")
        sid = lax.axis_index("s")
        wid = sid * NC + cid
        base_r = sid * RPT

        zeros16 = jnp.zeros((16,), jnp.float32)
        ones16 = jnp.ones((16,), jnp.float32)

        def fill_zero(r, carry):
            for cc in range(H // 16):
                ones_v[r, pl.ds(cc * 16, 16)] = zeros16
            return carry
        lax.fori_loop(0, CHUNK2, fill_zero, 0)

        for i in range(NZB):
            pltpu.sync_copy(ones_v.at[pl.ds(0, ZROWS)],
                            cnt_sh.at[pl.ds(base_r + i * ZROWS, ZROWS)])
        pltpu.sync_copy(ones_v.at[pl.ds(0, ZTAIL)],
                        cnt_sh.at[pl.ds(base_r + NZB * ZROWS, ZTAIL)])

        def fill_one(r, carry):
            for cc in range(H // 16):
                ones_v[r, pl.ds(cc * 16, 16)] = ones16
            return carry
        lax.fori_loop(0, CHUNK2, fill_one, 0)

        pltpu.sync_copy(dst_hbm.at[wid], dst_all)

        plsc.subcore_barrier()

        def body(j, carry):
            pltpu.sync_copy(ones_v, cnt_sh.at[dst_all.at[j]], add=True)
            return carry
        lax.fori_loop(0, tpc, body, 0)

        plsc.subcore_barrier()

        for i in range(NZB):
            r = base_r + i * ZROWS
            pltpu.sync_copy(cnt_sh.at[pl.ds(r, ZROWS)], ones_v.at[pl.ds(0, ZROWS)])
            pltpu.sync_copy(ones_v.at[pl.ds(0, ZROWS)], c_out.at[cid, pl.ds(r, ZROWS)])
        r = base_r + NZB * ZROWS
        pltpu.sync_copy(cnt_sh.at[pl.ds(r, ZTAIL)], ones_v.at[pl.ds(0, ZTAIL)])
        pltpu.sync_copy(ones_v.at[pl.ds(0, ZTAIL)], c_out.at[cid, pl.ds(r, ZTAIL)])

    return k(dst3)


def _tc_finish(sa, sb, ca, cb, W, b, gamma, beta):
    R = 1000

    def body(sa_ref, sb_ref, ca_ref, cb_ref, w_ref, b_ref, g_ref, be_ref, o_ref):
        s = sa_ref[...] + sb_ref[...]
        cnt = ca_ref[...] + cb_ref[...]
        y = lax.dot_general(s, w_ref[...], (((1,), (1,)), ((), ())),
                            preferred_element_type=jnp.float32)
        agg = (y + cnt * b_ref[...]) / (cnt + 1e-6)
        h = jnp.maximum(agg, 0.0)
        mean = jnp.mean(h, axis=1, keepdims=True)
        cen = h - mean
        var = jnp.mean(cen * cen, axis=1, keepdims=True)
        o_ref[...] = cen * lax.rsqrt(var + 1e-5) * g_ref[...] + be_ref[...]

    return pl.pallas_call(
        body,
        grid=(N_VAR // R,),
        in_specs=[
            pl.BlockSpec((R, H), lambda i: (i, 0)),
            pl.BlockSpec((R, H), lambda i: (i, 0)),
            pl.BlockSpec((R, H), lambda i: (i, 0)),
            pl.BlockSpec((R, H), lambda i: (i, 0)),
            pl.BlockSpec((H, H), lambda i: (0, 0)),
            pl.BlockSpec((1, H), lambda i: (0, 0)),
            pl.BlockSpec((1, H), lambda i: (0, 0)),
            pl.BlockSpec((1, H), lambda i: (0, 0)),
        ],
        out_specs=pl.BlockSpec((R, H), lambda i: (i, 0)),
        out_shape=jax.ShapeDtypeStruct((N_VAR, H), jnp.float32),
    )(sa, sb, ca, cb, W, b.reshape(1, H), gamma.reshape(1, H), beta.reshape(1, H))


def kernel(x_con, edge_index, num_var, W, b, gamma, beta):
    src = edge_index[0].astype(jnp.int32)
    dst = jnp.minimum(edge_index[1], num_var - 1).astype(jnp.int32)

    e = src.shape[0]
    e_pad = -e % (CHUNK * NW)
    src1 = jnp.concatenate([src, jnp.zeros((e_pad,), jnp.int32)]) if e_pad else src
    dst1 = jnp.concatenate([dst, jnp.full((e_pad,), N_PAD - 1, jnp.int32)]) if e_pad else dst
    tpc = (e + e_pad) // (CHUNK * NW)
    src3 = src1.reshape(NW, tpc, CHUNK)
    dst3 = dst1.reshape(NW, tpc, CHUNK)

    e_pad2 = -e % (CHUNK2 * NW)
    dst2 = jnp.concatenate([dst, jnp.full((e_pad2,), N_PAD - 1, jnp.int32)]) if e_pad2 else dst
    tpc2 = (e + e_pad2) // (CHUNK2 * NW)
    dst3b = dst2.reshape(NW, tpc2, CHUNK2)

    s_part = _sc_row_sums(x_con, src3, dst3, tpc)
    c_part = _sc_counts(dst3b, tpc2)
    return _tc_finish(s_part[0, :N_VAR], s_part[1, :N_VAR],
                      c_part[0, :N_VAR], c_part[1, :N_VAR],
                      W, b, gamma, beta)
